# baseline (speedup 1.0000x reference)
import functools

import jax
import jax.numpy as jnp
from jax import lax
from jax.experimental import pallas as pl
from jax.experimental.pallas import tpu as pltpu
from jax.experimental.pallas import tpu_sc as plsc

_B = 4
_N = 4096
_NC = 2
_NS = 16
_NW = _NC * _NS
_WPB = _NW // _B
_CHUNK = _N // _WPB
_L = 16
_QV = _CHUNK // _L


def _sc_chamfer(pos_t, xhat_t):
    mesh = plsc.VectorSubcoreMesh(core_axis_name="c", subcore_axis_name="s")

    @functools.partial(
        pl.kernel,
        mesh=mesh,
        out_type=jax.ShapeDtypeStruct((_NW, _L), jnp.float32),
        scratch_types=[
            pltpu.VMEM((3, _N), jnp.float32),
            pltpu.VMEM((3, _N), jnp.float32),
            pltpu.VMEM((_L,), jnp.float32),
        ],
    )
    def k(pos_hbm, xhat_hbm, out_hbm, a_ref, b_ref, o_ref):
        wid = lax.axis_index("s") * _NC + lax.axis_index("c")
        bat = wid // _WPB
        chk = wid % _WPB
        pltpu.sync_copy(pos_hbm.at[bat], a_ref)
        pltpu.sync_copy(xhat_hbm.at[bat], b_ref)

        def one_direction(q_ref, s_ref, acc0):
            def qblock(g, acc):
                qoff = chk * _CHUNK + g * _L
                qx = q_ref[0, pl.ds(qoff, _L)]
                qy = q_ref[1, pl.ds(qoff, _L)]
                qz = q_ref[2, pl.ds(qoff, _L)]

                def jloop(j, m):
                    soff = j * _L
                    sxv = s_ref[0, pl.ds(soff, _L)]
                    syv = s_ref[1, pl.ds(soff, _L)]
                    szv = s_ref[2, pl.ds(soff, _L)]
                    for e in range(_L):
                        dx = qx - sxv[e]
                        dy = qy - syv[e]
                        dz = qz - szv[e]
                        m = jnp.minimum(m, dx * dx + dy * dy + dz * dz)
                    return m

                m = lax.fori_loop(
                    0, _N // _L, jloop, jnp.full((_L,), 3.4e38, jnp.float32))
                return acc + m

            return lax.fori_loop(0, _QV, qblock, acc0)

        s = one_direction(a_ref, b_ref, jnp.zeros((_L,), jnp.float32))
        s = one_direction(b_ref, a_ref, s)
        o_ref[...] = s
        pltpu.sync_copy(o_ref, out_hbm.at[wid])

    return k(pos_t, xhat_t)


def kernel(pos, x_hat):
    pos_t = jnp.transpose(pos, (0, 2, 1))
    xhat_t = jnp.transpose(x_hat, (0, 2, 1))
    partial = _sc_chamfer(pos_t, xhat_t)
    return jnp.sum(partial) * jnp.float32(1.0 / (_B * _N))
